# Initial kernel scaffold; baseline (speedup 1.0000x reference)
import jax
import jax.numpy as jnp
from jax.experimental import pallas as pl


def kernel(x, permutations):
    raise NotImplementedError("write your pallas kernel here")


# speedup vs baseline: 1.3158x; 1.3158x over previous
import functools

import jax
import jax.numpy as jnp
from jax import lax
from jax.experimental import pallas as pl
from jax.experimental.pallas import tpu as pltpu
from jax.experimental.pallas import tpu_sc as plsc

L = 256
D = 32768
NC = 2
NS = 16
NW = NC * NS
RPW = L // NW


def _permute_body(x_hbm, perm_hbm, out_hbm, idx_ref, buf0, buf1,
                  gsem0, gsem1, ssem0, ssem1):
    c = lax.axis_index("c")
    s = lax.axis_index("s")
    wid = s * NC + c
    base = wid * RPW

    pltpu.sync_copy(perm_hbm.at[pl.ds(base, RPW)], idx_ref)

    bufs = (buf0, buf1)
    gsems = (gsem0, gsem1)
    ssems = (ssem0, ssem1)

    g = [None] * RPW
    st = [None] * RPW
    g[0] = pltpu.async_copy(x_hbm.at[idx_ref.at[0]], bufs[0], gsems[0])
    g[1] = pltpu.async_copy(x_hbm.at[idx_ref.at[1]], bufs[1], gsems[1])
    for k in range(RPW):
        sl = k % 2
        g[k].wait()
        st[k] = pltpu.async_copy(bufs[sl], out_hbm.at[pl.ds(base + k, 1)],
                                 ssems[sl])
        if k + 2 < RPW:
            st[k].wait()
            g[k + 2] = pltpu.async_copy(x_hbm.at[idx_ref.at[k + 2]],
                                        bufs[sl], gsems[sl])
    st[RPW - 2].wait()
    st[RPW - 1].wait()


@functools.partial(
    pl.kernel,
    out_type=jax.ShapeDtypeStruct((L, D), jnp.float32),
    mesh=plsc.VectorSubcoreMesh(core_axis_name="c", subcore_axis_name="s"),
    scratch_types=[
        pltpu.VMEM((RPW, 1), jnp.int32),
        pltpu.VMEM((1, D), jnp.float32),
        pltpu.VMEM((1, D), jnp.float32),
        pltpu.SemaphoreType.DMA,
        pltpu.SemaphoreType.DMA,
        pltpu.SemaphoreType.DMA,
        pltpu.SemaphoreType.DMA,
    ],
)
def _permute(x_hbm, perm_hbm, out_hbm, idx_ref, buf0, buf1,
             gsem0, gsem1, ssem0, ssem1):
    _permute_body(x_hbm, perm_hbm, out_hbm, idx_ref, buf0, buf1,
                  gsem0, gsem1, ssem0, ssem1)


def kernel(x, permutations):
    perm2d = permutations.astype(jnp.int32).reshape(L, 1)
    return _permute(x, perm2d)


# speedup vs baseline: 1.3359x; 1.0153x over previous
import functools

import jax
import jax.numpy as jnp
from jax import lax
from jax.experimental import pallas as pl
from jax.experimental.pallas import tpu as pltpu
from jax.experimental.pallas import tpu_sc as plsc

L = 256
D = 32768
NC = 2
NS = 16
NW = NC * NS
RPW = L // NW


NBUF = 3


def _permute_body(x_hbm, perm_hbm, out_hbm, idx_ref, bufs, gsems, ssems):
    c = lax.axis_index("c")
    s = lax.axis_index("s")
    wid = s * NC + c
    base = wid * RPW

    pltpu.sync_copy(perm_hbm.at[pl.ds(base, RPW)], idx_ref)

    g = [None] * RPW
    st = [None] * RPW
    for k in range(NBUF):
        g[k] = pltpu.async_copy(x_hbm.at[idx_ref.at[k]], bufs[k], gsems[k])
    for k in range(RPW):
        sl = k % NBUF
        g[k].wait()
        st[k] = pltpu.async_copy(bufs[sl], out_hbm.at[pl.ds(base + k, 1)],
                                 ssems[sl])
        if k + NBUF < RPW:
            st[k].wait()
            g[k + NBUF] = pltpu.async_copy(x_hbm.at[idx_ref.at[k + NBUF]],
                                           bufs[sl], gsems[sl])
    for k in range(RPW - NBUF, RPW):
        if st[k] is not None:
            st[k].wait()


@functools.partial(
    pl.kernel,
    out_type=jax.ShapeDtypeStruct((L, D), jnp.float32),
    mesh=plsc.VectorSubcoreMesh(core_axis_name="c", subcore_axis_name="s"),
    scratch_types=[
        pltpu.VMEM((RPW, 1), jnp.int32),
        [pltpu.VMEM((1, D), jnp.float32)] * NBUF,
        [pltpu.SemaphoreType.DMA] * NBUF,
        [pltpu.SemaphoreType.DMA] * NBUF,
    ],
)
def _permute(x_hbm, perm_hbm, out_hbm, idx_ref, bufs, gsems, ssems):
    _permute_body(x_hbm, perm_hbm, out_hbm, idx_ref, bufs, gsems, ssems)


def kernel(x, permutations):
    perm2d = permutations.astype(jnp.int32).reshape(L, 1)
    return _permute(x, perm2d)
